# baseline (speedup 1.0000x reference)
import functools

import jax
import jax.numpy as jnp
from jax import lax
from jax.experimental import pallas as pl
from jax.experimental.pallas import tpu as pltpu
from jax.experimental.pallas import tpu_sc as plsc

N_NODES = 100000
D_FEAT = 512
N_CLAMP = 16384
JBITS = 14

NC = 2
NS = 16
NW = NC * NS
RBASE = 3120
NBIG = (N_NODES - NW * RBASE) // 8
RMAX = RBASE + 8
NG = N_CLAMP // 16
MAPN = ((RMAX + 15) // 16) * 16
LISTN = MAPN + 16
WSTR = 2 * LISTN + 16
CH = 16
NBUF = 8

COPY_ROWS = 5000

_SC_MESH = dict(core_axis_name="c", subcore_axis_name="s",
                num_cores=NC, num_subcores=NS)


def _worker_range(wid):
    lo = pl.multiple_of(wid * RBASE + 8 * jnp.minimum(wid, NBIG), 8)
    hi = lo + RBASE + 8 * (wid < NBIG).astype(jnp.int32)
    return lo, hi


def _copy_body(src_ref, dst_ref):
    dst_ref[...] = src_ref[...]


@functools.cache
def _make_tc_copy():
    return pl.pallas_call(
        _copy_body,
        grid=(N_NODES // COPY_ROWS,),
        in_specs=[pl.BlockSpec((COPY_ROWS, D_FEAT), lambda i: (i, 0))],
        out_specs=pl.BlockSpec((COPY_ROWS, D_FEAT), lambda i: (i, 0)),
        out_shape=jax.ShapeDtypeStruct((N_NODES, D_FEAT), jnp.float32),
    )


def _build_body(idxs_hbm, values_hbm, pk_hbm, patch_hbm,
                idx_v, map_v, dst_v, srcj_v, tmp_v, *rest):
    bufs = rest[:NBUF]
    gsems = rest[NBUF:2 * NBUF]
    wsems = rest[2 * NBUF:3 * NBUF]
    wid = lax.axis_index("s") * NC + lax.axis_index("c")
    lo, hi = _worker_range(wid)

    pltpu.sync_copy(idxs_hbm, idx_v)

    def init_body(m, carry):
        map_v[pl.ds(pl.multiple_of(m * 16, 16), 16)] = jnp.full((16,), -1, jnp.int32)
        return carry
    lax.fori_loop(0, MAPN // 16, init_body, 0)

    iota = lax.iota(jnp.int32, 16)

    tmp_v[pl.ds(16, 16)] = jnp.full((16,), jnp.int32(0x7FFFFFFF))

    def scan_body(g, carry):
        jb = pl.multiple_of(g * 16, 16)
        idx = idx_v[pl.ds(jb, 16)]
        comp = idx * N_CLAMP + jb + iota
        skeys = jnp.sort(comp)
        tmp_v[pl.ds(0, 16)] = skeys
        nxt = tmp_v[pl.ds(1, 16)]
        row = lax.shift_right_logical(skeys, JBITS)
        nrow = lax.shift_right_logical(nxt, JBITS)
        mask = (row != nrow) & (row >= lo) & (row < hi)
        plsc.store_scatter(map_v, [row - lo], skeys, mask=mask)
        return carry
    lax.fori_loop(0, NG, scan_body, 0)

    def compact_body(m, cnt):
        off = pl.multiple_of(m * 16, 16)
        vec = map_v[pl.ds(off, 16)]
        msk = vec >= 0
        plsc.store_compressed(dst_v.at[pl.ds(cnt, 16)], off + iota + lo, mask=msk)
        plsc.store_compressed(srcj_v.at[pl.ds(cnt, 16)],
                              jnp.bitwise_and(vec, N_CLAMP - 1), mask=msk)
        return cnt + plsc.all_reduce_population_count(msk)[0]
    cnt = lax.fori_loop(0, MAPN // 16, compact_body, jnp.int32(0))

    @pl.when(cnt > 0)
    def _():
        last_d = dst_v[pl.ds(cnt - 1, 16)][0]
        last_j = srcj_v[pl.ds(cnt - 1, 16)][0]
        dst_v[pl.ds(cnt, 16)] = jnp.full((16,), jnp.int32(0)) + last_d
        srcj_v[pl.ds(cnt, 16)] = jnp.full((16,), jnp.int32(0)) + last_j

    tmp_v[pl.ds(0, 16)] = jnp.full((16,), jnp.int32(0)) + cnt

    base = pl.multiple_of(wid * WSTR, 8)
    pltpu.sync_copy(dst_v, pk_hbm.at[pl.ds(base, LISTN)])
    pltpu.sync_copy(srcj_v, pk_hbm.at[pl.ds(base + LISTN, LISTN)])
    pltpu.sync_copy(tmp_v.at[pl.ds(0, 16)],
                    pk_hbm.at[pl.ds(base + 2 * LISTN, 16)])

    prow = pl.multiple_of(wid * LISTN, 8)
    nch = lax.div(cnt + jnp.int32(CH - 1), jnp.int32(CH))

    def start_gather(k, b):
        jvec = srcj_v[pl.ds(k * CH, CH)]
        pltpu.async_copy(values_hbm.at[jvec], bufs[b], gsems[b])

    def start_write(k, b):
        pltpu.async_copy(bufs[b], patch_hbm.at[pl.ds(prow + k * CH, CH)],
                         wsems[b])

    def wait_gather(b):
        pltpu.make_async_copy(values_hbm.at[pl.ds(0, CH)], bufs[b],
                              gsems[b]).wait()

    def wait_write(b):
        pltpu.make_async_copy(values_hbm.at[pl.ds(0, CH)], bufs[b],
                              wsems[b]).wait()

    for b in range(NBUF):
        @pl.when(b < nch)
        def _(b=b):
            start_gather(jnp.int32(b), b)

    def outer(q, carry):
        kbase = q * NBUF
        for b in range(NBUF):
            k = kbase + b

            @pl.when(k < nch)
            def _(b=b, k=k):
                wait_gather(b)
                start_write(k, b)

                @pl.when(k + NBUF < nch)
                def _(b=b, k=k):
                    wait_write(b)
                    start_gather(k + NBUF, b)
        return carry
    nq = lax.div(nch + jnp.int32(NBUF - 1), jnp.int32(NBUF))
    lax.fori_loop(0, nq, outer, 0)

    for b in range(NBUF):
        @pl.when(b < nch)
        def _(b=b):
            wait_write(b)


@functools.cache
def _make_sc_build():
    return pl.kernel(
        _build_body,
        out_type=(
            jax.ShapeDtypeStruct((NW * WSTR,), jnp.int32),
            jax.ShapeDtypeStruct((NW * LISTN, D_FEAT), jnp.float32),
        ),
        mesh=plsc.VectorSubcoreMesh(**_SC_MESH),
        compiler_params=pltpu.CompilerParams(needs_layout_passes=False),
        scratch_types=(
            [pltpu.VMEM((N_CLAMP,), jnp.int32),
             pltpu.VMEM((MAPN,), jnp.int32),
             pltpu.VMEM((LISTN,), jnp.int32),
             pltpu.VMEM((LISTN,), jnp.int32),
             pltpu.VMEM((32,), jnp.int32)]
            + [pltpu.VMEM((CH, D_FEAT), jnp.float32)] * NBUF
            + [pltpu.SemaphoreType.DMA] * (2 * NBUF)
        ),
    )


def _apply_body(pk_hbm, patch_hbm, out_hbm, lists_v, *rest):
    bufs = rest[:NBUF]
    gsems = rest[NBUF:2 * NBUF]
    ssems = rest[2 * NBUF:3 * NBUF]

    wid = lax.axis_index("s") * NC + lax.axis_index("c")
    base = pl.multiple_of(wid * WSTR, 8)
    pltpu.sync_copy(pk_hbm.at[pl.ds(base, WSTR)], lists_v)
    cnt = lists_v[pl.ds(2 * LISTN, 16)][0]
    nch = lax.div(cnt + jnp.int32(CH - 1), jnp.int32(CH))

    prow = pl.multiple_of(wid * LISTN, 8)

    def start_gather(k, b):
        pltpu.async_copy(patch_hbm.at[pl.ds(prow + k * CH, CH)],
                         bufs[b], gsems[b])

    def start_scatter(k, b):
        dvec = lists_v[pl.ds(k * CH, CH)]
        pltpu.async_copy(bufs[b], out_hbm.at[dvec], ssems[b])

    def wait_gather(b):
        pltpu.make_async_copy(patch_hbm.at[pl.ds(0, CH)], bufs[b],
                              gsems[b]).wait()

    def wait_scatter(b):
        pltpu.make_async_copy(patch_hbm.at[pl.ds(0, CH)], bufs[b],
                              ssems[b]).wait()

    for b in range(NBUF):
        @pl.when(b < nch)
        def _(b=b):
            start_gather(jnp.int32(b), b)

    def outer(q, carry):
        kbase = q * NBUF
        for b in range(NBUF):
            k = kbase + b

            @pl.when(k < nch)
            def _(b=b, k=k):
                wait_gather(b)
                start_scatter(k, b)

                @pl.when(k + NBUF < nch)
                def _(b=b, k=k):
                    wait_scatter(b)
                    start_gather(k + NBUF, b)
        return carry
    nq = lax.div(nch + jnp.int32(NBUF - 1), jnp.int32(NBUF))
    lax.fori_loop(0, nq, outer, 0)

    for b in range(NBUF):
        @pl.when(b < nch)
        def _(b=b):
            wait_scatter(b)


@functools.cache
def _make_sc_apply():
    return pl.kernel(
        _apply_body,
        out_type=(),
        mesh=plsc.VectorSubcoreMesh(**_SC_MESH),
        compiler_params=pltpu.CompilerParams(needs_layout_passes=False),
        scratch_types=(
            [pltpu.VMEM((WSTR,), jnp.int32)]
            + [pltpu.VMEM((CH, D_FEAT), jnp.float32)] * NBUF
            + [pltpu.SemaphoreType.DMA] * (2 * NBUF)
        ),
    )


def kernel(nodes, idxs, values):
    packed, patch = _make_sc_build()(idxs, values)
    out = _make_tc_copy()(nodes)
    out_ref = jax.new_ref(out)
    _make_sc_apply()(packed, patch, out_ref)
    return out_ref[...]


# speedup vs baseline: 1.1045x; 1.1045x over previous
import functools

import jax
import jax.numpy as jnp
from jax import lax
from jax.experimental import pallas as pl
from jax.experimental.pallas import tpu as pltpu
from jax.experimental.pallas import tpu_sc as plsc

N_NODES = 100000
D_FEAT = 512
N_CLAMP = 16384
JBITS = 14

NC = 2
NS = 16
NW = NC * NS
RBASE = 3120
NBIG = (N_NODES - NW * RBASE) // 8
RMAX = RBASE + 8
NG = N_CLAMP // 16
MAPN = ((RMAX + 15) // 16) * 16
LISTN = MAPN + 16
WSTR = 2 * LISTN + 16
CH = 16
NBUF = 8

COPY_ROWS = 5000

_SC_MESH = dict(core_axis_name="c", subcore_axis_name="s",
                num_cores=NC, num_subcores=NS)


def _worker_range(wid):
    lo = pl.multiple_of(wid * RBASE + 8 * jnp.minimum(wid, NBIG), 8)
    hi = lo + RBASE + 8 * (wid < NBIG).astype(jnp.int32)
    return lo, hi


def _copy_body(src_ref, dst_ref):
    dst_ref[...] = src_ref[...]


@functools.cache
def _make_tc_copy():
    return pl.pallas_call(
        _copy_body,
        grid=(N_NODES // COPY_ROWS,),
        in_specs=[pl.BlockSpec((COPY_ROWS, D_FEAT), lambda i: (i, 0))],
        out_specs=pl.BlockSpec((COPY_ROWS, D_FEAT), lambda i: (i, 0)),
        out_shape=jax.ShapeDtypeStruct((N_NODES, D_FEAT), jnp.float32),
    )


def _build_body(idxs_hbm, pk_hbm,
                idx_v, map_v, dst_v, srcj_v, tmp_v):
    wid = lax.axis_index("s") * NC + lax.axis_index("c")
    lo, hi = _worker_range(wid)

    pltpu.sync_copy(idxs_hbm, idx_v)

    def init_body(m, carry):
        map_v[pl.ds(pl.multiple_of(m * 16, 16), 16)] = jnp.full((16,), -1, jnp.int32)
        return carry
    lax.fori_loop(0, MAPN // 16, init_body, 0)

    iota = lax.iota(jnp.int32, 16)

    tmp_v[pl.ds(16, 16)] = jnp.full((16,), jnp.int32(0x7FFFFFFF))

    def scan_body(g, carry):
        jb = pl.multiple_of(g * 16, 16)
        idx = idx_v[pl.ds(jb, 16)]
        comp = idx * N_CLAMP + jb + iota
        skeys = jnp.sort(comp)
        tmp_v[pl.ds(0, 16)] = skeys
        nxt = tmp_v[pl.ds(1, 16)]
        row = lax.shift_right_logical(skeys, JBITS)
        nrow = lax.shift_right_logical(nxt, JBITS)
        mask = (row != nrow) & (row >= lo) & (row < hi)
        plsc.store_scatter(map_v, [row - lo], skeys, mask=mask)
        return carry
    lax.fori_loop(0, NG, scan_body, 0)

    def compact_body(m, cnt):
        off = pl.multiple_of(m * 16, 16)
        vec = map_v[pl.ds(off, 16)]
        msk = vec >= 0
        plsc.store_compressed(dst_v.at[pl.ds(cnt, 16)], off + iota + lo, mask=msk)
        plsc.store_compressed(srcj_v.at[pl.ds(cnt, 16)],
                              jnp.bitwise_and(vec, N_CLAMP - 1), mask=msk)
        return cnt + plsc.all_reduce_population_count(msk)[0]
    cnt = lax.fori_loop(0, MAPN // 16, compact_body, jnp.int32(0))

    @pl.when(cnt > 0)
    def _():
        last_d = dst_v[pl.ds(cnt - 1, 16)][0]
        last_j = srcj_v[pl.ds(cnt - 1, 16)][0]
        dst_v[pl.ds(cnt, 16)] = jnp.full((16,), jnp.int32(0)) + last_d
        srcj_v[pl.ds(cnt, 16)] = jnp.full((16,), jnp.int32(0)) + last_j

    tmp_v[pl.ds(0, 16)] = jnp.full((16,), jnp.int32(0)) + cnt

    base = pl.multiple_of(wid * WSTR, 8)
    pltpu.sync_copy(dst_v, pk_hbm.at[pl.ds(base, LISTN)])
    pltpu.sync_copy(srcj_v, pk_hbm.at[pl.ds(base + LISTN, LISTN)])
    pltpu.sync_copy(tmp_v.at[pl.ds(0, 16)],
                    pk_hbm.at[pl.ds(base + 2 * LISTN, 16)])


@functools.cache
def _make_sc_build():
    return pl.kernel(
        _build_body,
        out_type=jax.ShapeDtypeStruct((NW * WSTR,), jnp.int32),
        mesh=plsc.VectorSubcoreMesh(**_SC_MESH),
        compiler_params=pltpu.CompilerParams(needs_layout_passes=False),
        scratch_types=[
            pltpu.VMEM((N_CLAMP,), jnp.int32),
            pltpu.VMEM((MAPN,), jnp.int32),
            pltpu.VMEM((LISTN,), jnp.int32),
            pltpu.VMEM((LISTN,), jnp.int32),
            pltpu.VMEM((32,), jnp.int32),
        ],
    )


def _apply_body(pk_hbm, values_hbm, out_hbm, lists_v, *rest):
    bufs = rest[:NBUF]
    gsems = rest[NBUF:2 * NBUF]
    ssems = rest[2 * NBUF:3 * NBUF]

    wid = lax.axis_index("s") * NC + lax.axis_index("c")
    base = pl.multiple_of(wid * WSTR, 8)
    pltpu.sync_copy(pk_hbm.at[pl.ds(base, WSTR)], lists_v)
    cnt = lists_v[pl.ds(2 * LISTN, 16)][0]
    nch = lax.div(cnt + jnp.int32(CH - 1), jnp.int32(CH))

    def start_gather(k, b):
        jvec = lists_v[pl.ds(LISTN + k * CH, CH)]
        pltpu.async_copy(values_hbm.at[jvec], bufs[b], gsems[b])

    def start_scatter(k, b):
        dvec = lists_v[pl.ds(k * CH, CH)]
        pltpu.async_copy(bufs[b], out_hbm.at[dvec], ssems[b])

    def wait_gather(b):
        pltpu.make_async_copy(values_hbm.at[pl.ds(0, CH)], bufs[b],
                              gsems[b]).wait()

    def wait_scatter(b):
        pltpu.make_async_copy(values_hbm.at[pl.ds(0, CH)], bufs[b],
                              ssems[b]).wait()

    for b in range(NBUF):
        @pl.when(b < nch)
        def _(b=b):
            start_gather(jnp.int32(b), b)

    def outer(q, carry):
        kbase = q * NBUF
        for b in range(NBUF):
            k = kbase + b

            @pl.when(k < nch)
            def _(b=b, k=k):
                wait_gather(b)
                start_scatter(k, b)

                @pl.when(k + NBUF < nch)
                def _(b=b, k=k):
                    wait_scatter(b)
                    start_gather(k + NBUF, b)
        return carry
    nq = lax.div(nch + jnp.int32(NBUF - 1), jnp.int32(NBUF))
    lax.fori_loop(0, nq, outer, 0)

    for b in range(NBUF):
        @pl.when(b < nch)
        def _(b=b):
            wait_scatter(b)


@functools.cache
def _make_sc_apply():
    return pl.kernel(
        _apply_body,
        out_type=(),
        mesh=plsc.VectorSubcoreMesh(**_SC_MESH),
        compiler_params=pltpu.CompilerParams(needs_layout_passes=False),
        scratch_types=(
            [pltpu.VMEM((WSTR,), jnp.int32)]
            + [pltpu.VMEM((CH, D_FEAT), jnp.float32)] * NBUF
            + [pltpu.SemaphoreType.DMA] * (2 * NBUF)
        ),
    )


def kernel(nodes, idxs, values):
    packed = _make_sc_build()(idxs)
    out = _make_tc_copy()(nodes)
    out_ref = jax.new_ref(out)
    _make_sc_apply()(packed, values, out_ref)
    return out_ref[...]


# speedup vs baseline: 1.1046x; 1.0001x over previous
import functools

import jax
import jax.numpy as jnp
from jax import lax
from jax.experimental import pallas as pl
from jax.experimental.pallas import tpu as pltpu
from jax.experimental.pallas import tpu_sc as plsc

N_NODES = 100000
D_FEAT = 512
N_CLAMP = 16384
JBITS = 14

NC = 2
NS = 16
NW = NC * NS
RBASE = 3120
NBIG = (N_NODES - NW * RBASE) // 8
RMAX = RBASE + 8
NG = N_CLAMP // 16
MAPN = ((RMAX + 15) // 16) * 16
LISTN = MAPN + 16
WSTR = 2 * LISTN + 16
CH = 16
NBUF = 8

COPY_ROWS = 4000

_SC_MESH = dict(core_axis_name="c", subcore_axis_name="s",
                num_cores=NC, num_subcores=NS)


def _worker_range(wid):
    lo = pl.multiple_of(wid * RBASE + 8 * jnp.minimum(wid, NBIG), 8)
    hi = lo + RBASE + 8 * (wid < NBIG).astype(jnp.int32)
    return lo, hi


def _copy_body(src_ref, dst_ref):
    dst_ref[...] = src_ref[...]


@functools.cache
def _make_tc_copy():
    return pl.pallas_call(
        _copy_body,
        grid=(N_NODES // COPY_ROWS,),
        in_specs=[pl.BlockSpec((COPY_ROWS, D_FEAT), lambda i: (i, 0))],
        out_specs=pl.BlockSpec((COPY_ROWS, D_FEAT), lambda i: (i, 0)),
        out_shape=jax.ShapeDtypeStruct((N_NODES, D_FEAT), jnp.float32),
    )


def _build_body(idxs_hbm, pk_hbm,
                idx_v, map_v, dst_v, srcj_v, tmp_v):
    wid = lax.axis_index("s") * NC + lax.axis_index("c")
    lo, hi = _worker_range(wid)

    pltpu.sync_copy(idxs_hbm, idx_v)

    def init_body(m, carry):
        map_v[pl.ds(pl.multiple_of(m * 16, 16), 16)] = jnp.full((16,), -1, jnp.int32)
        return carry
    lax.fori_loop(0, MAPN // 16, init_body, 0)

    iota = lax.iota(jnp.int32, 16)

    tmp_v[pl.ds(16, 16)] = jnp.full((16,), jnp.int32(0x7FFFFFFF))

    def scan_body(g, carry):
        jb = pl.multiple_of(g * 16, 16)
        idx = idx_v[pl.ds(jb, 16)]
        comp = idx * N_CLAMP + jb + iota
        skeys = jnp.sort(comp)
        tmp_v[pl.ds(0, 16)] = skeys
        nxt = tmp_v[pl.ds(1, 16)]
        row = lax.shift_right_logical(skeys, JBITS)
        nrow = lax.shift_right_logical(nxt, JBITS)
        mask = (row != nrow) & (row >= lo) & (row < hi)
        plsc.store_scatter(map_v, [row - lo], skeys, mask=mask)
        return carry
    lax.fori_loop(0, NG, scan_body, 0)

    def compact_body(m, cnt):
        off = pl.multiple_of(m * 16, 16)
        vec = map_v[pl.ds(off, 16)]
        msk = vec >= 0
        plsc.store_compressed(dst_v.at[pl.ds(cnt, 16)], off + iota + lo, mask=msk)
        plsc.store_compressed(srcj_v.at[pl.ds(cnt, 16)],
                              jnp.bitwise_and(vec, N_CLAMP - 1), mask=msk)
        return cnt + plsc.all_reduce_population_count(msk)[0]
    cnt = lax.fori_loop(0, MAPN // 16, compact_body, jnp.int32(0))

    @pl.when(cnt > 0)
    def _():
        last_d = dst_v[pl.ds(cnt - 1, 16)][0]
        last_j = srcj_v[pl.ds(cnt - 1, 16)][0]
        dst_v[pl.ds(cnt, 16)] = jnp.full((16,), jnp.int32(0)) + last_d
        srcj_v[pl.ds(cnt, 16)] = jnp.full((16,), jnp.int32(0)) + last_j

    tmp_v[pl.ds(0, 16)] = jnp.full((16,), jnp.int32(0)) + cnt

    base = pl.multiple_of(wid * WSTR, 8)
    pltpu.sync_copy(dst_v, pk_hbm.at[pl.ds(base, LISTN)])
    pltpu.sync_copy(srcj_v, pk_hbm.at[pl.ds(base + LISTN, LISTN)])
    pltpu.sync_copy(tmp_v.at[pl.ds(0, 16)],
                    pk_hbm.at[pl.ds(base + 2 * LISTN, 16)])


@functools.cache
def _make_sc_build():
    return pl.kernel(
        _build_body,
        out_type=jax.ShapeDtypeStruct((NW * WSTR,), jnp.int32),
        mesh=plsc.VectorSubcoreMesh(**_SC_MESH),
        compiler_params=pltpu.CompilerParams(needs_layout_passes=False),
        scratch_types=[
            pltpu.VMEM((N_CLAMP,), jnp.int32),
            pltpu.VMEM((MAPN,), jnp.int32),
            pltpu.VMEM((LISTN,), jnp.int32),
            pltpu.VMEM((LISTN,), jnp.int32),
            pltpu.VMEM((32,), jnp.int32),
        ],
    )


def _apply_body(pk_hbm, values_hbm, out_hbm, lists_v, *rest):
    bufs = rest[:NBUF]
    gsems = rest[NBUF:2 * NBUF]
    ssems = rest[2 * NBUF:3 * NBUF]

    wid = lax.axis_index("s") * NC + lax.axis_index("c")
    base = pl.multiple_of(wid * WSTR, 8)
    pltpu.sync_copy(pk_hbm.at[pl.ds(base, WSTR)], lists_v)
    cnt = lists_v[pl.ds(2 * LISTN, 16)][0]
    nch = lax.div(cnt + jnp.int32(CH - 1), jnp.int32(CH))

    def start_gather(k, b):
        jvec = lists_v[pl.ds(LISTN + k * CH, CH)]
        pltpu.async_copy(values_hbm.at[jvec], bufs[b], gsems[b])

    def start_scatter(k, b):
        dvec = lists_v[pl.ds(k * CH, CH)]
        pltpu.async_copy(bufs[b], out_hbm.at[dvec], ssems[b])

    def wait_gather(b):
        pltpu.make_async_copy(values_hbm.at[pl.ds(0, CH)], bufs[b],
                              gsems[b]).wait()

    def wait_scatter(b):
        pltpu.make_async_copy(values_hbm.at[pl.ds(0, CH)], bufs[b],
                              ssems[b]).wait()

    for b in range(NBUF):
        @pl.when(b < nch)
        def _(b=b):
            start_gather(jnp.int32(b), b)

    def outer(q, carry):
        kbase = q * NBUF
        for b in range(NBUF):
            k = kbase + b

            @pl.when(k < nch)
            def _(b=b, k=k):
                wait_gather(b)
                start_scatter(k, b)

                @pl.when(k + NBUF < nch)
                def _(b=b, k=k):
                    wait_scatter(b)
                    start_gather(k + NBUF, b)
        return carry
    nq = lax.div(nch + jnp.int32(NBUF - 1), jnp.int32(NBUF))
    lax.fori_loop(0, nq, outer, 0)

    for b in range(NBUF):
        @pl.when(b < nch)
        def _(b=b):
            wait_scatter(b)


@functools.cache
def _make_sc_apply():
    return pl.kernel(
        _apply_body,
        out_type=(),
        mesh=plsc.VectorSubcoreMesh(**_SC_MESH),
        compiler_params=pltpu.CompilerParams(needs_layout_passes=False),
        scratch_types=(
            [pltpu.VMEM((WSTR,), jnp.int32)]
            + [pltpu.VMEM((CH, D_FEAT), jnp.float32)] * NBUF
            + [pltpu.SemaphoreType.DMA] * (2 * NBUF)
        ),
    )


def kernel(nodes, idxs, values):
    packed = _make_sc_build()(idxs)
    out = _make_tc_copy()(nodes)
    out_ref = jax.new_ref(out)
    _make_sc_apply()(packed, values, out_ref)
    return out_ref[...]


# speedup vs baseline: 1.1052x; 1.0005x over previous
import functools

import jax
import jax.numpy as jnp
from jax import lax
from jax.experimental import pallas as pl
from jax.experimental.pallas import tpu as pltpu
from jax.experimental.pallas import tpu_sc as plsc

N_NODES = 100000
D_FEAT = 512
N_CLAMP = 16384
JBITS = 14

NC = 2
NS = 16
NW = NC * NS
RBASE = 3120
NBIG = (N_NODES - NW * RBASE) // 8
RMAX = RBASE + 8
NG = N_CLAMP // 16
MAPN = ((RMAX + 15) // 16) * 16
LISTN = MAPN + 16
WSTR = 2 * LISTN + 16
CH = 16
NBUF = 12

COPY_ROWS = 5000

_SC_MESH = dict(core_axis_name="c", subcore_axis_name="s",
                num_cores=NC, num_subcores=NS)


def _worker_range(wid):
    lo = pl.multiple_of(wid * RBASE + 8 * jnp.minimum(wid, NBIG), 8)
    hi = lo + RBASE + 8 * (wid < NBIG).astype(jnp.int32)
    return lo, hi


def _copy_body(src_ref, dst_ref):
    dst_ref[...] = src_ref[...]


@functools.cache
def _make_tc_copy():
    return pl.pallas_call(
        _copy_body,
        grid=(N_NODES // COPY_ROWS,),
        in_specs=[pl.BlockSpec((COPY_ROWS, D_FEAT), lambda i: (i, 0))],
        out_specs=pl.BlockSpec((COPY_ROWS, D_FEAT), lambda i: (i, 0)),
        out_shape=jax.ShapeDtypeStruct((N_NODES, D_FEAT), jnp.float32),
    )


def _build_body(idxs_hbm, pk_hbm,
                idx_v, map_v, dst_v, srcj_v, tmp_v):
    wid = lax.axis_index("s") * NC + lax.axis_index("c")
    lo, hi = _worker_range(wid)

    pltpu.sync_copy(idxs_hbm, idx_v)

    def init_body(m, carry):
        map_v[pl.ds(pl.multiple_of(m * 16, 16), 16)] = jnp.full((16,), -1, jnp.int32)
        return carry
    lax.fori_loop(0, MAPN // 16, init_body, 0)

    iota = lax.iota(jnp.int32, 16)

    tmp_v[pl.ds(16, 16)] = jnp.full((16,), jnp.int32(0x7FFFFFFF))

    def scan_body(g, carry):
        jb = pl.multiple_of(g * 16, 16)
        idx = idx_v[pl.ds(jb, 16)]
        comp = idx * N_CLAMP + jb + iota
        skeys = jnp.sort(comp)
        tmp_v[pl.ds(0, 16)] = skeys
        nxt = tmp_v[pl.ds(1, 16)]
        row = lax.shift_right_logical(skeys, JBITS)
        nrow = lax.shift_right_logical(nxt, JBITS)
        mask = (row != nrow) & (row >= lo) & (row < hi)
        plsc.store_scatter(map_v, [row - lo], skeys, mask=mask)
        return carry
    lax.fori_loop(0, NG, scan_body, 0)

    def compact_body(m, cnt):
        off = pl.multiple_of(m * 16, 16)
        vec = map_v[pl.ds(off, 16)]
        msk = vec >= 0
        plsc.store_compressed(dst_v.at[pl.ds(cnt, 16)], off + iota + lo, mask=msk)
        plsc.store_compressed(srcj_v.at[pl.ds(cnt, 16)],
                              jnp.bitwise_and(vec, N_CLAMP - 1), mask=msk)
        return cnt + plsc.all_reduce_population_count(msk)[0]
    cnt = lax.fori_loop(0, MAPN // 16, compact_body, jnp.int32(0))

    @pl.when(cnt > 0)
    def _():
        last_d = dst_v[pl.ds(cnt - 1, 16)][0]
        last_j = srcj_v[pl.ds(cnt - 1, 16)][0]
        dst_v[pl.ds(cnt, 16)] = jnp.full((16,), jnp.int32(0)) + last_d
        srcj_v[pl.ds(cnt, 16)] = jnp.full((16,), jnp.int32(0)) + last_j

    tmp_v[pl.ds(0, 16)] = jnp.full((16,), jnp.int32(0)) + cnt

    base = pl.multiple_of(wid * WSTR, 8)
    pltpu.sync_copy(dst_v, pk_hbm.at[pl.ds(base, LISTN)])
    pltpu.sync_copy(srcj_v, pk_hbm.at[pl.ds(base + LISTN, LISTN)])
    pltpu.sync_copy(tmp_v.at[pl.ds(0, 16)],
                    pk_hbm.at[pl.ds(base + 2 * LISTN, 16)])


@functools.cache
def _make_sc_build():
    return pl.kernel(
        _build_body,
        out_type=jax.ShapeDtypeStruct((NW * WSTR,), jnp.int32),
        mesh=plsc.VectorSubcoreMesh(**_SC_MESH),
        compiler_params=pltpu.CompilerParams(needs_layout_passes=False),
        scratch_types=[
            pltpu.VMEM((N_CLAMP,), jnp.int32),
            pltpu.VMEM((MAPN,), jnp.int32),
            pltpu.VMEM((LISTN,), jnp.int32),
            pltpu.VMEM((LISTN,), jnp.int32),
            pltpu.VMEM((32,), jnp.int32),
        ],
    )


def _apply_body(pk_hbm, values_hbm, out_hbm, lists_v, *rest):
    bufs = rest[:NBUF]
    gsems = rest[NBUF:2 * NBUF]
    ssems = rest[2 * NBUF:3 * NBUF]

    wid = lax.axis_index("s") * NC + lax.axis_index("c")
    base = pl.multiple_of(wid * WSTR, 8)
    pltpu.sync_copy(pk_hbm.at[pl.ds(base, WSTR)], lists_v)
    cnt = lists_v[pl.ds(2 * LISTN, 16)][0]
    nch = lax.div(cnt + jnp.int32(CH - 1), jnp.int32(CH))

    def start_gather(k, b):
        jvec = lists_v[pl.ds(LISTN + k * CH, CH)]
        pltpu.async_copy(values_hbm.at[jvec], bufs[b], gsems[b])

    def start_scatter(k, b):
        dvec = lists_v[pl.ds(k * CH, CH)]
        pltpu.async_copy(bufs[b], out_hbm.at[dvec], ssems[b])

    def wait_gather(b):
        pltpu.make_async_copy(values_hbm.at[pl.ds(0, CH)], bufs[b],
                              gsems[b]).wait()

    def wait_scatter(b):
        pltpu.make_async_copy(values_hbm.at[pl.ds(0, CH)], bufs[b],
                              ssems[b]).wait()

    for b in range(NBUF):
        @pl.when(b < nch)
        def _(b=b):
            start_gather(jnp.int32(b), b)

    def outer(q, carry):
        kbase = q * NBUF
        for b in range(NBUF):
            k = kbase + b

            @pl.when(k < nch)
            def _(b=b, k=k):
                wait_gather(b)
                start_scatter(k, b)

                @pl.when(k + NBUF < nch)
                def _(b=b, k=k):
                    wait_scatter(b)
                    start_gather(k + NBUF, b)
        return carry
    nq = lax.div(nch + jnp.int32(NBUF - 1), jnp.int32(NBUF))
    lax.fori_loop(0, nq, outer, 0)

    for b in range(NBUF):
        @pl.when(b < nch)
        def _(b=b):
            wait_scatter(b)


@functools.cache
def _make_sc_apply():
    return pl.kernel(
        _apply_body,
        out_type=(),
        mesh=plsc.VectorSubcoreMesh(**_SC_MESH),
        compiler_params=pltpu.CompilerParams(needs_layout_passes=False),
        scratch_types=(
            [pltpu.VMEM((WSTR,), jnp.int32)]
            + [pltpu.VMEM((CH, D_FEAT), jnp.float32)] * NBUF
            + [pltpu.SemaphoreType.DMA] * (2 * NBUF)
        ),
    )


def kernel(nodes, idxs, values):
    packed = _make_sc_build()(idxs)
    out = _make_tc_copy()(nodes)
    out_ref = jax.new_ref(out)
    _make_sc_apply()(packed, values, out_ref)
    return out_ref[...]


# speedup vs baseline: 1.1065x; 1.0012x over previous
import functools

import jax
import jax.numpy as jnp
from jax import lax
from jax.experimental import pallas as pl
from jax.experimental.pallas import tpu as pltpu
from jax.experimental.pallas import tpu_sc as plsc

N_NODES = 100000
D_FEAT = 512
N_CLAMP = 16384
JBITS = 14

NC = 2
NS = 16
NW = NC * NS
RBASE = 3120
NBIG = (N_NODES - NW * RBASE) // 8
RMAX = RBASE + 8
NG = N_CLAMP // 16
MAPN = ((RMAX + 15) // 16) * 16
LISTN = MAPN + 16
WSTR = 2 * LISTN + 16
CH = 16
NBUF = 8

COPY_ROWS = 5000

_SC_MESH = dict(core_axis_name="c", subcore_axis_name="s",
                num_cores=NC, num_subcores=NS)


def _worker_range(wid):
    lo = pl.multiple_of(wid * RBASE + 8 * jnp.minimum(wid, NBIG), 8)
    hi = lo + RBASE + 8 * (wid < NBIG).astype(jnp.int32)
    return lo, hi


def _copy_body(src_ref, dst_ref):
    dst_ref[...] = src_ref[...]


@functools.cache
def _make_tc_copy():
    return pl.pallas_call(
        _copy_body,
        grid=(N_NODES // COPY_ROWS,),
        in_specs=[pl.BlockSpec((COPY_ROWS, D_FEAT), lambda i: (i, 0))],
        out_specs=pl.BlockSpec((COPY_ROWS, D_FEAT), lambda i: (i, 0)),
        out_shape=jax.ShapeDtypeStruct((N_NODES, D_FEAT), jnp.float32),
    )


def _build_body(idxs_hbm, pk_hbm,
                idx_v, map_v, dst_v, srcj_v, tmp_v):
    wid = lax.axis_index("s") * NC + lax.axis_index("c")
    lo, hi = _worker_range(wid)

    pltpu.sync_copy(idxs_hbm, idx_v)

    def init_body(m, carry):
        map_v[pl.ds(pl.multiple_of(m * 16, 16), 16)] = jnp.full((16,), -1, jnp.int32)
        return carry
    lax.fori_loop(0, MAPN // 16, init_body, 0)

    iota = lax.iota(jnp.int32, 16)

    tmp_v[pl.ds(16, 16)] = jnp.full((16,), jnp.int32(0x7FFFFFFF))

    def scan_body(g, carry):
        jb = pl.multiple_of(g * 16, 16)
        idx = idx_v[pl.ds(jb, 16)]
        comp = idx * N_CLAMP + jb + iota
        skeys = jnp.sort(comp)
        tmp_v[pl.ds(0, 16)] = skeys
        nxt = tmp_v[pl.ds(1, 16)]
        row = lax.shift_right_logical(skeys, JBITS)
        nrow = lax.shift_right_logical(nxt, JBITS)
        mask = (row != nrow) & (row >= lo) & (row < hi)
        plsc.store_scatter(map_v, [row - lo], skeys, mask=mask)
        return carry
    lax.fori_loop(0, NG, scan_body, 0)

    def compact_body(m, cnt):
        off = pl.multiple_of(m * 16, 16)
        vec = map_v[pl.ds(off, 16)]
        msk = vec >= 0
        plsc.store_compressed(dst_v.at[pl.ds(cnt, 16)], off + iota + lo, mask=msk)
        plsc.store_compressed(srcj_v.at[pl.ds(cnt, 16)],
                              jnp.bitwise_and(vec, N_CLAMP - 1), mask=msk)
        return cnt + plsc.all_reduce_population_count(msk)[0]
    cnt = lax.fori_loop(0, MAPN // 16, compact_body, jnp.int32(0))

    @pl.when(cnt > 0)
    def _():
        last_d = dst_v[pl.ds(cnt - 1, 16)][0]
        last_j = srcj_v[pl.ds(cnt - 1, 16)][0]
        dst_v[pl.ds(cnt, 16)] = jnp.full((16,), jnp.int32(0)) + last_d
        srcj_v[pl.ds(cnt, 16)] = jnp.full((16,), jnp.int32(0)) + last_j

    tmp_v[pl.ds(0, 16)] = jnp.full((16,), jnp.int32(0)) + cnt

    base = pl.multiple_of(wid * WSTR, 8)
    pltpu.sync_copy(dst_v, pk_hbm.at[pl.ds(base, LISTN)])
    pltpu.sync_copy(srcj_v, pk_hbm.at[pl.ds(base + LISTN, LISTN)])
    pltpu.sync_copy(tmp_v.at[pl.ds(0, 16)],
                    pk_hbm.at[pl.ds(base + 2 * LISTN, 16)])


@functools.cache
def _make_sc_build():
    return pl.kernel(
        _build_body,
        out_type=jax.ShapeDtypeStruct((NW * WSTR,), jnp.int32),
        mesh=plsc.VectorSubcoreMesh(**_SC_MESH),
        compiler_params=pltpu.CompilerParams(needs_layout_passes=False),
        scratch_types=[
            pltpu.VMEM((N_CLAMP,), jnp.int32),
            pltpu.VMEM((MAPN,), jnp.int32),
            pltpu.VMEM((LISTN,), jnp.int32),
            pltpu.VMEM((LISTN,), jnp.int32),
            pltpu.VMEM((32,), jnp.int32),
        ],
    )


def _apply_body(pk_hbm, values_hbm, out_hbm, lists_v, *rest):
    bufs = rest[:NBUF]
    gsems = rest[NBUF:2 * NBUF]
    ssems = rest[2 * NBUF:3 * NBUF]

    wid = lax.axis_index("s") * NC + lax.axis_index("c")
    base = pl.multiple_of(wid * WSTR, 8)
    pltpu.sync_copy(pk_hbm.at[pl.ds(base, WSTR)], lists_v)
    cnt = lists_v[pl.ds(2 * LISTN, 16)][0]
    nch = lax.div(cnt + jnp.int32(CH - 1), jnp.int32(CH))

    def start_gather(k, b):
        jvec = lists_v[pl.ds(LISTN + k * CH, CH)]
        pltpu.async_copy(values_hbm.at[jvec], bufs[b], gsems[b])

    def start_scatter(k, b):
        dvec = lists_v[pl.ds(k * CH, CH)]
        pltpu.async_copy(bufs[b], out_hbm.at[dvec], ssems[b])

    def wait_gather(b):
        pltpu.make_async_copy(values_hbm.at[pl.ds(0, CH)], bufs[b],
                              gsems[b]).wait()

    def wait_scatter(b):
        pltpu.make_async_copy(values_hbm.at[pl.ds(0, CH)], bufs[b],
                              ssems[b]).wait()

    for b in range(NBUF):
        @pl.when(b < nch)
        def _(b=b):
            start_gather(jnp.int32(b), b)

    def outer(q, carry):
        kbase = q * NBUF
        for b in range(NBUF):
            k = kbase + b

            @pl.when(k < nch)
            def _(b=b, k=k):
                wait_gather(b)
                start_scatter(k, b)

                @pl.when(k + NBUF < nch)
                def _(b=b, k=k):
                    wait_scatter(b)
                    start_gather(k + NBUF, b)
        return carry
    nq = lax.div(nch + jnp.int32(NBUF - 1), jnp.int32(NBUF))
    lax.fori_loop(0, nq, outer, 0)

    for b in range(NBUF):
        @pl.when(b < nch)
        def _(b=b):
            wait_scatter(b)


@functools.cache
def _make_sc_apply():
    return pl.kernel(
        _apply_body,
        out_type=(),
        mesh=plsc.VectorSubcoreMesh(**_SC_MESH),
        compiler_params=pltpu.CompilerParams(needs_layout_passes=False),
        scratch_types=(
            [pltpu.VMEM((WSTR,), jnp.int32)]
            + [pltpu.VMEM((CH, D_FEAT), jnp.float32)] * NBUF
            + [pltpu.SemaphoreType.DMA] * (2 * NBUF)
        ),
    )


def kernel(nodes, idxs, values):
    packed = _make_sc_build()(idxs)
    out = _make_tc_copy()(nodes)
    out_ref = jax.new_ref(out)
    _make_sc_apply()(packed, values, out_ref)
    return out_ref[...]
